# baseline (speedup 1.0000x reference)
import jax
import jax.numpy as jnp
from jax.experimental import pallas as pl

_C = 81
_TOP_K = 200
_KEEP = 200
_CT = 0.5
_NT = 0.45


def _decode(loc, priors, variances):
    cxcy = priors[:, :2] + loc[:, :2] * variances[:, :2] * priors[:, 2:]
    wh = priors[:, 2:] * jnp.exp(loc[:, 2:] * variances[:, 2:])
    mins = cxcy - wh / 2.0
    maxs = mins + wh
    return jnp.concatenate([mins, maxs], axis=1)


def _pairwise_iou(boxes):
    x1, y1, x2, y2 = boxes[:, 0], boxes[:, 1], boxes[:, 2], boxes[:, 3]
    area = jnp.clip(x2 - x1, 0.0) * jnp.clip(y2 - y1, 0.0)
    ix1 = jnp.maximum(x1[:, None], x1[None, :])
    iy1 = jnp.maximum(y1[:, None], y1[None, :])
    ix2 = jnp.minimum(x2[:, None], x2[None, :])
    iy2 = jnp.minimum(y2[:, None], y2[None, :])
    iw = jnp.clip(ix2 - ix1, 0.0)
    ih = jnp.clip(iy2 - iy1, 0.0)
    inter = iw * ih
    union = area[:, None] + area[None, :] - inter
    return inter / jnp.maximum(union, 1e-12)


def _nms_class(boxes, scores):
    n = boxes.shape[0]
    iou = jax.lax.stop_gradient(_pairwise_iou(boxes))
    valid = scores > -jnp.inf
    idxs = jnp.arange(n)

    def body(i, keep):
        suppressed = jnp.any((idxs < i) & keep & (iou[:, i] > _NT))
        return keep.at[i].set(valid[i] & jnp.logical_not(suppressed))

    keep = jax.lax.fori_loop(0, n, body, jnp.zeros((n,), dtype=bool))
    return keep


def _ident(x_ref, o_ref):
    o_ref[...] = x_ref[...]


def kernel(loc_data, conf_data, prior_data):
    batch = loc_data.shape[0]
    num_priors = loc_data.shape[1] // 4
    loc = loc_data.reshape(batch, num_priors, 4)
    conf = conf_data.reshape(batch, num_priors, _C)
    pv = prior_data.reshape(2, num_priors, 4)
    priors, variances = pv[0], pv[1]

    cls_ids = jnp.array([c for c in range(_C) if c != 0], dtype=jnp.int32)

    def per_image(loc_i, conf_i, img_idx):
        decoded = _decode(loc_i, priors, variances)

        def per_class(cl):
            s = conf_i[:, cl]
            s = jnp.where(s > _CT, s, -jnp.inf)
            top_s, top_i = jax.lax.top_k(s, _TOP_K)
            cand = decoded[top_i]
            keep = _nms_class(cand, top_s)
            det_s = jnp.where(keep, top_s, -jnp.inf)
            return det_s, cand

        det_s, cand = jax.vmap(per_class)(cls_ids)
        labels = jnp.broadcast_to(cls_ids[:, None].astype(jnp.float32), det_s.shape)
        flat_s = det_s.reshape(-1)
        flat_b = cand.reshape(-1, 4)
        flat_l = labels.reshape(-1)
        order = jnp.argsort(-flat_s)[:_KEEP]
        sel_s = flat_s[order]
        sel_b = flat_b[order]
        sel_l = flat_l[order]
        valid = sel_s > -jnp.inf
        img_col = jnp.broadcast_to(img_idx, (_KEEP,))
        rows = jnp.concatenate(
            [img_col[:, None], sel_l[:, None], sel_s[:, None], sel_b], axis=1
        )
        rows = jnp.where(valid[:, None], rows, 0.0)
        return rows

    out = jax.vmap(per_image)(loc, conf, jnp.arange(batch, dtype=jnp.float32))
    out = out[:, None, :, :]
    return pl.pallas_call(
        _ident, out_shape=jax.ShapeDtypeStruct(out.shape, out.dtype)
    )(out)


# speedup vs baseline: 4.5194x; 4.5194x over previous
import jax
import jax.numpy as jnp
from jax import lax
from jax.experimental import pallas as pl
from jax.experimental.pallas import tpu as pltpu
from jax.experimental.pallas import tpu_sc as plsc

B = 8
P = 20000
C = 81
FG = 80
NTASK = B * FG
K = 200
CT_BITS = 0x3F000000
NMS_T = 0.45
NC = 2
NS = 16
NW = NC * NS
TPW = NTASK // NW

_MESH = plsc.VectorSubcoreMesh(core_axis_name="c", subcore_axis_name="s")
_CP = pltpu.CompilerParams(needs_layout_passes=False)


def _wid():
    return lax.axis_index("s") * NC + lax.axis_index("c")


def _iota():
    return lax.iota(jnp.int32, 16)


def _extract(vec, lane):
    return jnp.sum(jnp.where(_iota() == lane, vec, jnp.zeros_like(vec)))


def _splat(scalar):
    return jnp.full((16,), scalar)


def _phase_a_body(conf_hbm, locf_hbm, pv_hbm, keys_hbm, table_hbm,
                  slab, colbuf, locslab, pslab, vslab, stage, sem):
    iota = _iota()
    wid = _wid()
    b = wid // 4
    q = wid % 4
    start = jnp.where(q < 2, q * 5008, 10016 + (q - 2) * 4992)
    count = jnp.where(q < 2, 5008, 4992)
    CH = 400

    @pl.loop(0, 13)
    def _chunk(j):
        cs = pl.multiple_of(
            jnp.where(j < 12, start + j * CH, start + count - CH), 16)
        off = pl.multiple_of((b * P + cs) * C, 16)
        pltpu.sync_copy(conf_hbm.at[pl.ds(off, CH * C)], slab)

        @pl.loop(0, FG)
        def _cls(cf):
            cl = cf + 1
            for v in range(CH // 16):
                idx = (iota + v * 16) * C + cl
                val = plsc.load_gather(slab, [idx])
                colbuf[pl.ds(cf * CH + v * 16, 16)] = plsc.bitcast(
                    val, jnp.int32)

        @pl.loop(0, FG)
        def _fire(cf):
            doff = pl.multiple_of((b * FG + cf) * P + cs, 16)
            dst = keys_hbm.at[pl.ds(doff, CH)]
            pltpu.async_copy(colbuf.at[pl.ds(cf * CH, CH)], dst, sem)

        pltpu.make_async_copy(
            keys_hbm.at[pl.ds(0, FG * CH)], colbuf, sem).wait()

    CH2 = 752

    @pl.loop(0, 7)
    def _tchunk(j):
        cs = pl.multiple_of(
            jnp.where(j < 6, start + j * CH2, start + count - CH2), 16)
        pltpu.sync_copy(
            locf_hbm.at[pl.ds(pl.multiple_of((b * P + cs) * 4, 16),
                              CH2 * 4)], locslab)
        pltpu.sync_copy(
            pv_hbm.at[pl.ds(pl.multiple_of(cs * 4, 16), CH2 * 4)], pslab)
        pltpu.sync_copy(
            pv_hbm.at[pl.ds(pl.multiple_of((P + cs) * 4, 16), CH2 * 4)],
            vslab)

        @pl.loop(0, CH2 // 16)
        def _pack(v):
            pidx = v * 16 + iota
            for comp in range(4):
                lv = plsc.load_gather(locslab, [pidx * 4 + comp])
                plsc.store_scatter(stage, [pidx * 16 + comp], lv)
                pv_ = plsc.load_gather(pslab, [pidx * 4 + comp])
                plsc.store_scatter(stage, [pidx * 16 + 4 + comp], pv_)
                vv = plsc.load_gather(vslab, [pidx * 4 + comp])
                plsc.store_scatter(stage, [pidx * 16 + 8 + comp], vv)

        pltpu.sync_copy(
            stage,
            table_hbm.at[pl.ds(pl.multiple_of((b * P + cs) * 16, 16),
                               CH2 * 16)])


def _find_thresh(hist, gsum, target):
    iota = _iota()
    zero16 = jnp.zeros((16,), jnp.int32)

    @pl.loop(0, 64)
    def _red(g):
        acc = zero16
        for l in range(16):
            acc = acc + hist[pl.ds(l * 1024 + g * 16, 16)]
            hist[pl.ds(l * 1024 + g * 16, 16)] = zero16
        gsum[pl.ds(g * 16, 16)] = acc

    def body(i, carry):
        acc, bt, found = carry
        g = 63 - i
        v = gsum[pl.ds(g * 16, 16)]
        rv = lax.rev(v, (0,))
        cs = plsc.cumsum(rv) + acc
        reach = cs >= target
        nreach = jnp.sum(reach.astype(jnp.int32))
        any_ = nreach > 0
        j = 16 - nreach
        c15 = _extract(cs, 15)
        take = any_ & (found == 0)
        bt = jnp.where(take, g * 16 + 15 - j, bt)
        found = jnp.where(any_, 1, found)
        acc = jnp.where(found == 1, acc, c15)
        return acc, bt, found

    _, bt, _ = lax.fori_loop(
        0, 64, body, (jnp.int32(0), jnp.int32(-1), jnp.int32(0)))
    return bt


def _list_level(shift, mask, srck, srcp, n_src, ns, selk, selp,
                hist, gsum, bk, bp, target):
    iota = _iota()
    ones16 = jnp.ones((16,), jnp.int32)
    nv = (n_src + 15) >> 4

    def hbody(j, carry):
        tail = (j * 16 + iota) < n_src
        kv = srck[pl.ds(j * 16, 16)]
        bucket = (kv >> shift) & mask
        plsc.addupdate_scatter(hist, [iota * 1024 + bucket], ones16,
                               mask=tail)
        return carry

    lax.fori_loop(0, nv, hbody, 0)
    bt = _find_thresh(hist, gsum, target)

    def cbody(j, carry):
        ns_, nb_ = carry
        tail = (j * 16 + iota) < n_src
        kv = srck[pl.ds(j * 16, 16)]
        pv = srcp[pl.ds(j * 16, 16)]
        bucket = (kv >> shift) & mask
        sm = tail & (bucket > bt)
        bm = tail & (bucket == bt)
        plsc.store_compressed(selk.at[pl.ds(ns_, 16)], kv, mask=sm)
        plsc.store_compressed(selp.at[pl.ds(ns_, 16)], pv, mask=sm)
        plsc.store_compressed(bk.at[pl.ds(nb_, 16)], kv, mask=bm)
        plsc.store_compressed(bp.at[pl.ds(nb_, 16)], pv, mask=bm)
        return (ns_ + jnp.sum(sm.astype(jnp.int32)),
                nb_ + jnp.sum(bm.astype(jnp.int32)))

    ns2, nb2 = lax.fori_loop(0, nv, cbody, (ns, jnp.int32(0)))
    return ns2, nb2


def _select_topk(src, nvec, hist, gsum, selk, selp, bak, bap, bbk, bbp):
    iota = _iota()
    ones16 = jnp.ones((16,), jnp.int32)

    def hbody(j, carry):
        kv = src[pl.ds(j * 16, 16)]
        m = kv > CT_BITS
        bucket = (kv >> 13) & 1023
        plsc.addupdate_scatter(hist, [iota * 1024 + bucket], ones16, mask=m)
        return carry

    lax.fori_loop(0, nvec, hbody, 0)
    bt1 = _find_thresh(hist, gsum, jnp.int32(K))

    def cbody(j, carry):
        ns_, nb_ = carry
        kv = src[pl.ds(j * 16, 16)]
        m = kv > CT_BITS
        bucket = (kv >> 13) & 1023
        sm = m & (bucket > bt1)
        bm = m & (bucket == bt1)
        pv = j * 16 + iota
        plsc.store_compressed(selk.at[pl.ds(ns_, 16)], kv, mask=sm)
        plsc.store_compressed(selp.at[pl.ds(ns_, 16)], pv, mask=sm)
        plsc.store_compressed(bak.at[pl.ds(nb_, 16)], kv, mask=bm)
        plsc.store_compressed(bap.at[pl.ds(nb_, 16)], pv, mask=bm)
        return (ns_ + jnp.sum(sm.astype(jnp.int32)),
                nb_ + jnp.sum(bm.astype(jnp.int32)))

    ns, nb1 = lax.fori_loop(0, nvec, cbody, (jnp.int32(0), jnp.int32(0)))

    ns, nb2 = _list_level(3, 1023, bak, bap, nb1, ns, selk, selp,
                          hist, gsum, bbk, bbp, jnp.int32(K) - ns)
    ns, nb3 = _list_level(0, 7, bbk, bbp, nb2, ns, selk, selp,
                          hist, gsum, bak, bap, jnp.int32(K) - ns)

    need = jnp.minimum(jnp.int32(K) - ns, nb3)

    def tbody(j, ns_):
        take = (j * 16 + iota) < need
        kv = bak[pl.ds(j * 16, 16)]
        pv = bap[pl.ds(j * 16, 16)]
        plsc.store_compressed(selk.at[pl.ds(ns_, 16)], kv, mask=take)
        plsc.store_compressed(selp.at[pl.ds(ns_, 16)], pv, mask=take)
        return ns_ + jnp.sum(take.astype(jnp.int32))

    ns = lax.fori_loop(0, (need + 15) >> 4, tbody, ns)
    return ns


def _rank_sort(selk, selp, n_sel, skey, spidx):
    iota = _iota()
    zero16 = jnp.zeros((16,), jnp.int32)
    for g in range(13):
        skey[pl.ds(g * 16, 16)] = zero16
        spidx[pl.ds(g * 16, 16)] = zero16
    ng = (n_sel + 15) >> 4
    lane0 = iota == 0

    def jbody(j, carry):
        gj = j >> 4
        lj = j & 15
        kv = selk[pl.ds(gj * 16, 16)]
        pvv = selp[pl.ds(gj * 16, 16)]
        kj = _extract(kv, lj)
        pj = _extract(pvv, lj)
        kjv = _splat(kj)
        pjv = _splat(pj)

        def gbody(g, rank):
            ks = selk[pl.ds(g * 16, 16)]
            ps = selp[pl.ds(g * 16, 16)]
            validm = (g * 16 + iota) < n_sel
            gt = (ks > kjv) | ((ks == kjv) & (ps < pjv))
            return rank + jnp.sum((gt & validm).astype(jnp.int32))

        rank = lax.fori_loop(0, ng, gbody, jnp.int32(0))
        rv = _splat(rank)
        plsc.store_scatter(skey, [rv], kjv, mask=lane0)
        plsc.store_scatter(spidx, [rv], pjv, mask=lane0)
        return carry

    lax.fori_loop(0, n_sel, jbody, 0)


def _phase_b1_body(keys_hbm, skeys_hbm, spids_hbm,
                   sbuf, hist, gsum, selk, selp, bak, bap, bbk, bbp,
                   skey, spidx):
    zero16 = jnp.zeros((16,), jnp.int32)

    @pl.loop(0, 1024)
    def _zh(i):
        hist[pl.ds(i * 16, 16)] = zero16

    @pl.loop(0, TPW)
    def _task(t):
        task = _wid() * TPW + t
        pltpu.sync_copy(keys_hbm.at[pl.ds(pl.multiple_of(task * P, 16), P)],
                        sbuf)
        n_sel = _select_topk(sbuf, P // 16, hist, gsum, selk, selp,
                             bak, bap, bbk, bbp)
        _rank_sort(selk, selp, n_sel, skey, spidx)
        pltpu.sync_copy(skey.at[pl.ds(0, K)],
                        skeys_hbm.at[pl.ds(pl.multiple_of(task * K, 8), K)])
        pltpu.sync_copy(spidx.at[pl.ds(0, K)],
                        spids_hbm.at[pl.ds(pl.multiple_of(task * K, 8), K)])


def _phase_b2_body(skeys_hbm, spids_hbm, table2, dets_hbm, detb_hbm,
                   skey, spidx, grows, bx1, by1, bx2, by2, barea,
                   supp, detk, dstage, ia, ib, sem):
    iota = _iota()
    zero16 = jnp.zeros((16,), jnp.int32)
    zero16f = jnp.zeros((16,), jnp.float32)

    @pl.loop(0, TPW)
    def _task(t):
        task = _wid() * TPW + t
        b = task // FG
        skey[pl.ds(192, 16)] = zero16
        spidx[pl.ds(192, 16)] = zero16
        pltpu.sync_copy(skeys_hbm.at[pl.ds(pl.multiple_of(task * K, 8), K)],
                        skey.at[pl.ds(0, K)])
        pltpu.sync_copy(spids_hbm.at[pl.ds(pl.multiple_of(task * K, 8), K)],
                        spidx.at[pl.ds(0, K)])

        n_sel = jnp.int32(0)
        for g in range(13):
            n_sel = n_sel + jnp.sum(
                (skey[pl.ds(g * 16, 16)] > CT_BITS).astype(jnp.int32))

        base = b * P
        for v in range(8):
            sp = spidx[pl.ds(v * 16, 16)]
            ia[pl.ds(v * 16, 16)] = (base + sp) >> 3
        for v in range(5):
            sp = spidx[pl.ds(128 + v * 16, 16)]
            ib[pl.ds(v * 16, 16)] = (base + sp) >> 3
        d1 = pltpu.async_copy(table2.at[ia], grows.at[pl.ds(0, 128)], sem)
        d2 = pltpu.async_copy(table2.at[ib], grows.at[pl.ds(128, 80)], sem)
        d1.wait()
        d2.wait()

        for g in range(13):
            rg = g * 16 + iota
            sp = spidx[pl.ds(g * 16, 16)]
            colb = (sp & 7) * 16
            comps = [plsc.load_gather(grows, [rg, colb + cc])
                     for cc in range(12)]
            l0, l1, l2, l3, p0, p1, p2, p3, v0, v1, v2, v3 = comps
            cx = p0 + l0 * v0 * p2
            cy = p1 + l1 * v1 * p3
            w = p2 * jnp.exp(l2 * v2)
            h = p3 * jnp.exp(l3 * v3)
            x1 = cx - w / 2.0
            y1 = cy - h / 2.0
            x2 = x1 + w
            y2 = y1 + h
            bx1[pl.ds(g * 16, 16)] = x1
            by1[pl.ds(g * 16, 16)] = y1
            bx2[pl.ds(g * 16, 16)] = x2
            by2[pl.ds(g * 16, 16)] = y2
            barea[pl.ds(g * 16, 16)] = (
                jnp.maximum(x2 - x1, 0.0) * jnp.maximum(y2 - y1, 0.0))
            supp[pl.ds(g * 16, 16)] = zero16f
            detk[pl.ds(g * 16, 16)] = zero16

        ng = (n_sel + 15) >> 4

        def ibody(i, carry):
            gi = i >> 4
            li = i & 15
            si = _extract(supp[pl.ds(gi * 16, 16)], li)
            keep = si == 0.0
            ki = _extract(skey[pl.ds(gi * 16, 16)], li)
            plsc.store_scatter(detk, [_splat(i)], _splat(ki),
                               mask=(iota == 0) & keep)

            @pl.when(keep)
            def _sweep():
                xx1 = _splat(_extract(bx1[pl.ds(gi * 16, 16)], li))
                yy1 = _splat(_extract(by1[pl.ds(gi * 16, 16)], li))
                xx2 = _splat(_extract(bx2[pl.ds(gi * 16, 16)], li))
                yy2 = _splat(_extract(by2[pl.ds(gi * 16, 16)], li))
                aai = _splat(_extract(barea[pl.ds(gi * 16, 16)], li))

                def gbody(g, cc):
                    ax1 = bx1[pl.ds(g * 16, 16)]
                    ay1 = by1[pl.ds(g * 16, 16)]
                    ax2 = bx2[pl.ds(g * 16, 16)]
                    ay2 = by2[pl.ds(g * 16, 16)]
                    aar = barea[pl.ds(g * 16, 16)]
                    ix1 = jnp.maximum(ax1, xx1)
                    iy1 = jnp.maximum(ay1, yy1)
                    ix2 = jnp.minimum(ax2, xx2)
                    iy2 = jnp.minimum(ay2, yy2)
                    iw = jnp.maximum(ix2 - ix1, 0.0)
                    ih = jnp.maximum(iy2 - iy1, 0.0)
                    inter = iw * ih
                    union = aar + aai - inter
                    q = inter / jnp.maximum(union, 1e-12)
                    hit = (q > NMS_T).astype(jnp.float32)
                    sg = supp[pl.ds(g * 16, 16)]
                    supp[pl.ds(g * 16, 16)] = jnp.maximum(sg, hit)
                    return cc

                lax.fori_loop(gi, ng, gbody, 0)

            return carry

        lax.fori_loop(0, n_sel, ibody, 0)

        for g in range(13):
            rg = g * 16 + iota
            for comp, ref in ((0, bx1), (1, by1), (2, bx2), (3, by2)):
                plsc.store_scatter(dstage, [rg * 16 + comp],
                                   ref[pl.ds(g * 16, 16)])

        pltpu.sync_copy(detk.at[pl.ds(0, K)],
                        dets_hbm.at[pl.ds(pl.multiple_of(task * K, 8), K)])
        pltpu.sync_copy(
            dstage.at[pl.ds(0, K * 16)],
            detb_hbm.at[pl.ds(pl.multiple_of(task * K * 16, 16), K * 16)])


def _phase_c_body(dets_hbm, detb2, out_hbm,
                  sbuf, hist, gsum, selk, selp, bak, bap, bbk, bbp,
                  skey, spidx, rowbuf, grows, ia, ib, sem):
    iota = _iota()
    zero16 = jnp.zeros((16,), jnp.int32)
    zero16f = jnp.zeros((16,), jnp.float32)
    wid = _wid()

    @pl.loop(0, 1024)
    def _zh(i):
        hist[pl.ds(i * 16, 16)] = zero16

    @pl.when(wid < B)
    def _img():
        b = wid
        pltpu.sync_copy(
            dets_hbm.at[pl.ds(pl.multiple_of(b * FG * K, 16), FG * K)],
            sbuf)
        n_sel = _select_topk(sbuf, FG * K // 16, hist, gsum, selk, selp,
                             bak, bap, bbk, bbp)
        _rank_sort(selk, selp, n_sel, skey, spidx)

        base = b * FG * K
        for v in range(8):
            sp = spidx[pl.ds(v * 16, 16)]
            ia[pl.ds(v * 16, 16)] = (base + sp) >> 3
        for v in range(5):
            sp = spidx[pl.ds(128 + v * 16, 16)]
            ib[pl.ds(v * 16, 16)] = (base + sp) >> 3
        d1 = pltpu.async_copy(detb2.at[ia], grows.at[pl.ds(0, 128)], sem)
        d2 = pltpu.async_copy(detb2.at[ib], grows.at[pl.ds(128, 80)], sem)
        d1.wait()
        d2.wait()

        for g in range(88):
            rowbuf[pl.ds(g * 16, 16)] = zero16f

        bf = _splat(jnp.float32(b))
        for g in range(13):
            rg = g * 16 + iota
            valid = rg < n_sel
            kv = skey[pl.ds(g * 16, 16)]
            sp = spidx[pl.ds(g * 16, 16)]
            score = plsc.bitcast(kv, jnp.float32)
            label = (sp // K + 1).astype(jnp.float32)
            r7 = rg * 7
            plsc.store_scatter(rowbuf, [r7], bf, mask=valid)
            plsc.store_scatter(rowbuf, [r7 + 1], label, mask=valid)
            plsc.store_scatter(rowbuf, [r7 + 2], score, mask=valid)
            colb = (sp & 7) * 16
            for comp in range(4):
                bv = plsc.load_gather(grows, [rg, colb + comp])
                plsc.store_scatter(rowbuf, [r7 + 3 + comp], bv, mask=valid)

        pltpu.sync_copy(
            rowbuf.at[pl.ds(0, K * 7)],
            out_hbm.at[pl.ds(pl.multiple_of(b * K * 7, 8), K * 7)])


def kernel(loc_data, conf_data, prior_data):
    conf_flat = conf_data.reshape(B * P * C)
    loc_flat = loc_data.reshape(B * P * 4)
    pv_flat = prior_data.reshape(2 * P * 4)

    phase_a = pl.kernel(
        _phase_a_body,
        out_type=(
            jax.ShapeDtypeStruct((NTASK * P,), jnp.int32),
            jax.ShapeDtypeStruct((B * P * 16,), jnp.float32),
        ),
        mesh=_MESH,
        compiler_params=_CP,
        scratch_types=[
            pltpu.VMEM((400 * C,), jnp.float32),
            pltpu.VMEM((FG * 400,), jnp.int32),
            pltpu.VMEM((752 * 4,), jnp.float32),
            pltpu.VMEM((752 * 4,), jnp.float32),
            pltpu.VMEM((752 * 4,), jnp.float32),
            pltpu.VMEM((752 * 16,), jnp.float32),
            pltpu.SemaphoreType.DMA,
        ],
    )
    keys, table = phase_a(conf_flat, loc_flat, pv_flat)

    phase_b1 = pl.kernel(
        _phase_b1_body,
        out_type=(
            jax.ShapeDtypeStruct((NTASK * K,), jnp.int32),
            jax.ShapeDtypeStruct((NTASK * K,), jnp.int32),
        ),
        mesh=_MESH,
        compiler_params=_CP,
        scratch_types=[
            pltpu.VMEM((P,), jnp.int32),
            pltpu.VMEM((16 * 1024,), jnp.int32),
            pltpu.VMEM((1024,), jnp.int32),
            pltpu.VMEM((256,), jnp.int32),
            pltpu.VMEM((256,), jnp.int32),
            pltpu.VMEM((P,), jnp.int32),
            pltpu.VMEM((P,), jnp.int32),
            pltpu.VMEM((P,), jnp.int32),
            pltpu.VMEM((P,), jnp.int32),
            pltpu.VMEM((208,), jnp.int32),
            pltpu.VMEM((208,), jnp.int32),
        ],
    )
    skeys, spids = phase_b1(keys)

    phase_b2 = pl.kernel(
        _phase_b2_body,
        out_type=(
            jax.ShapeDtypeStruct((NTASK * K,), jnp.int32),
            jax.ShapeDtypeStruct((NTASK * K * 16,), jnp.float32),
        ),
        mesh=_MESH,
        compiler_params=_CP,
        scratch_types=[
            pltpu.VMEM((208,), jnp.int32),
            pltpu.VMEM((208,), jnp.int32),
            pltpu.VMEM((208, 128), jnp.float32),
            pltpu.VMEM((208,), jnp.float32),
            pltpu.VMEM((208,), jnp.float32),
            pltpu.VMEM((208,), jnp.float32),
            pltpu.VMEM((208,), jnp.float32),
            pltpu.VMEM((208,), jnp.float32),
            pltpu.VMEM((208,), jnp.float32),
            pltpu.VMEM((208,), jnp.int32),
            pltpu.VMEM((208 * 16,), jnp.float32),
            pltpu.VMEM((128,), jnp.int32),
            pltpu.VMEM((80,), jnp.int32),
            pltpu.SemaphoreType.DMA,
        ],
    )
    dets, detb = phase_b2(
        skeys, spids, table.reshape(B * P // 8, 128))

    phase_c = pl.kernel(
        _phase_c_body,
        out_type=jax.ShapeDtypeStruct((B * K * 7,), jnp.float32),
        mesh=_MESH,
        compiler_params=_CP,
        scratch_types=[
            pltpu.VMEM((FG * K,), jnp.int32),
            pltpu.VMEM((16 * 1024,), jnp.int32),
            pltpu.VMEM((1024,), jnp.int32),
            pltpu.VMEM((256,), jnp.int32),
            pltpu.VMEM((256,), jnp.int32),
            pltpu.VMEM((FG * K,), jnp.int32),
            pltpu.VMEM((FG * K,), jnp.int32),
            pltpu.VMEM((FG * K,), jnp.int32),
            pltpu.VMEM((FG * K,), jnp.int32),
            pltpu.VMEM((208,), jnp.int32),
            pltpu.VMEM((208,), jnp.int32),
            pltpu.VMEM((88 * 16,), jnp.float32),
            pltpu.VMEM((208, 128), jnp.float32),
            pltpu.VMEM((128,), jnp.int32),
            pltpu.VMEM((80,), jnp.int32),
            pltpu.SemaphoreType.DMA,
        ],
    )
    out = phase_c(dets, detb.reshape(NTASK * K // 8, 128))
    return out.reshape(B, 1, K, 7)


# speedup vs baseline: 4.7476x; 1.0505x over previous
import jax
import jax.numpy as jnp
from jax import lax
from jax.experimental import pallas as pl
from jax.experimental.pallas import tpu as pltpu
from jax.experimental.pallas import tpu_sc as plsc

B = 8
P = 20000
C = 81
FG = 80
NTASK = B * FG
K = 200
CT_BITS = 0x3F000000
NMS_T = 0.45
NC = 2
NS = 16
NW = NC * NS
TPW = NTASK // NW

_MESH = plsc.VectorSubcoreMesh(core_axis_name="c", subcore_axis_name="s")
_CP = pltpu.CompilerParams(needs_layout_passes=False)

_GDN = lax.GatherDimensionNumbers(
    offset_dims=(), collapsed_slice_dims=(0,), start_index_map=(0,))


def _wid():
    return lax.axis_index("s") * NC + lax.axis_index("c")


def _iota():
    return lax.iota(jnp.int32, 16)


def _extract(vec, lane):
    return jnp.sum(jnp.where(_iota() == lane, vec, jnp.zeros_like(vec)))


def _splat_lane(vec, lane):
    return lax.gather(vec, jnp.full((16, 1), lane, jnp.int32), _GDN, (1,),
                      mode=lax.GatherScatterMode.PROMISE_IN_BOUNDS)


def _splat(scalar):
    return jnp.full((16,), scalar)


def _phase_a_body(conf_hbm, locf_hbm, pv_hbm, keys_hbm, table_hbm,
                  slab, colbuf, locslab, pslab, vslab, stage, sem):
    iota = _iota()
    wid = _wid()
    b = wid // 4
    q = wid % 4
    start = jnp.where(q < 2, q * 5008, 10016 + (q - 2) * 4992)
    count = jnp.where(q < 2, 5008, 4992)
    CH = 400

    @pl.loop(0, 13)
    def _chunk(j):
        cs = pl.multiple_of(
            jnp.where(j < 12, start + j * CH, start + count - CH), 16)
        off = pl.multiple_of((b * P + cs) * C, 16)
        pltpu.sync_copy(conf_hbm.at[pl.ds(off, CH * C)], slab)

        @pl.loop(0, FG)
        def _cls(cf):
            cl = cf + 1
            for v in range(CH // 16):
                idx = (iota + v * 16) * C + cl
                val = plsc.load_gather(slab, [idx])
                colbuf[pl.ds(cf * CH + v * 16, 16)] = plsc.bitcast(
                    val, jnp.int32)

        @pl.loop(0, FG)
        def _fire(cf):
            doff = pl.multiple_of((b * FG + cf) * P + cs, 16)
            dst = keys_hbm.at[pl.ds(doff, CH)]
            pltpu.async_copy(colbuf.at[pl.ds(cf * CH, CH)], dst, sem)

        pltpu.make_async_copy(
            keys_hbm.at[pl.ds(0, FG * CH)], colbuf, sem).wait()

    CH2 = 752

    @pl.loop(0, 7)
    def _tchunk(j):
        cs = pl.multiple_of(
            jnp.where(j < 6, start + j * CH2, start + count - CH2), 16)
        pltpu.sync_copy(
            locf_hbm.at[pl.ds(pl.multiple_of((b * P + cs) * 4, 16),
                              CH2 * 4)], locslab)
        pltpu.sync_copy(
            pv_hbm.at[pl.ds(pl.multiple_of(cs * 4, 16), CH2 * 4)], pslab)
        pltpu.sync_copy(
            pv_hbm.at[pl.ds(pl.multiple_of((P + cs) * 4, 16), CH2 * 4)],
            vslab)

        @pl.loop(0, CH2 // 16)
        def _pack(v):
            pidx = v * 16 + iota
            for comp in range(4):
                lv = plsc.load_gather(locslab, [pidx * 4 + comp])
                plsc.store_scatter(stage, [pidx * 16 + comp], lv)
                pv_ = plsc.load_gather(pslab, [pidx * 4 + comp])
                plsc.store_scatter(stage, [pidx * 16 + 4 + comp], pv_)
                vv = plsc.load_gather(vslab, [pidx * 4 + comp])
                plsc.store_scatter(stage, [pidx * 16 + 8 + comp], vv)

        pltpu.sync_copy(
            stage,
            table_hbm.at[pl.ds(pl.multiple_of((b * P + cs) * 16, 16),
                               CH2 * 16)])


def _find_thresh(hist, gsum, target):
    iota = _iota()
    zero16 = jnp.zeros((16,), jnp.int32)

    @pl.loop(0, 64)
    def _red(g):
        acc = zero16
        for l in range(16):
            acc = acc + hist[pl.ds(l * 1024 + g * 16, 16)]
            hist[pl.ds(l * 1024 + g * 16, 16)] = zero16
        gsum[pl.ds(g * 16, 16)] = acc

    def body(i, carry):
        acc, bt, found = carry
        g = 63 - i
        v = gsum[pl.ds(g * 16, 16)]
        rv = lax.rev(v, (0,))
        cs = plsc.cumsum(rv) + acc
        reach = cs >= target
        nreach = jnp.sum(reach.astype(jnp.int32))
        any_ = nreach > 0
        j = 16 - nreach
        c15 = _extract(cs, 15)
        take = any_ & (found == 0)
        bt = jnp.where(take, g * 16 + 15 - j, bt)
        found = jnp.where(any_, 1, found)
        acc = jnp.where(found == 1, acc, c15)
        return acc, bt, found

    _, bt, _ = lax.fori_loop(
        0, 64, body, (jnp.int32(0), jnp.int32(-1), jnp.int32(0)))
    return bt


def _list_level(shift, mask, bk, bp, n_src, ns, selk, selp,
                hist, gsum, target):
    iota = _iota()
    ones16 = jnp.ones((16,), jnp.int32)
    nv = (n_src + 15) >> 4

    def hbody(j, carry):
        tail = (j * 16 + iota) < n_src
        kv = bk[pl.ds(j * 16, 16)]
        bucket = (kv >> shift) & mask
        plsc.addupdate_scatter(hist, [iota * 1024 + bucket], ones16,
                               mask=tail)
        return carry

    lax.fori_loop(0, nv, hbody, 0)
    bt = _find_thresh(hist, gsum, target)

    def cbody(j, carry):
        ns_, nb_ = carry
        tail = (j * 16 + iota) < n_src
        kv = bk[pl.ds(j * 16, 16)]
        pv = bp[pl.ds(j * 16, 16)]
        bucket = (kv >> shift) & mask
        sm = tail & (bucket > bt)
        bm = tail & (bucket == bt)
        plsc.store_compressed(selk.at[pl.ds(ns_, 16)], kv, mask=sm)
        plsc.store_compressed(selp.at[pl.ds(ns_, 16)], pv, mask=sm)
        plsc.store_compressed(bk.at[pl.ds(nb_, 16)], kv, mask=bm)
        plsc.store_compressed(bp.at[pl.ds(nb_, 16)], pv, mask=bm)
        return (ns_ + jnp.sum(sm.astype(jnp.int32)),
                nb_ + jnp.sum(bm.astype(jnp.int32)))

    ns2, nb2 = lax.fori_loop(0, nv, cbody, (ns, jnp.int32(0)))
    return ns2, nb2


def _select_topk(src, nvec, hist, gsum, selk, selp, bak, bap):
    iota = _iota()
    ones16 = jnp.ones((16,), jnp.int32)

    def hbody(j, carry):
        kv = src[pl.ds(j * 16, 16)]
        m = kv > CT_BITS
        bucket = (kv >> 13) & 1023
        plsc.addupdate_scatter(hist, [iota * 1024 + bucket], ones16, mask=m)
        return carry

    lax.fori_loop(0, nvec, hbody, 0)
    bt1 = _find_thresh(hist, gsum, jnp.int32(K))

    def cbody(j, carry):
        ns_, nb_ = carry
        kv = src[pl.ds(j * 16, 16)]
        m = kv > CT_BITS
        bucket = (kv >> 13) & 1023
        sm = m & (bucket > bt1)
        bm = m & (bucket == bt1)
        pv = j * 16 + iota
        plsc.store_compressed(selk.at[pl.ds(ns_, 16)], kv, mask=sm)
        plsc.store_compressed(selp.at[pl.ds(ns_, 16)], pv, mask=sm)
        plsc.store_compressed(bak.at[pl.ds(nb_, 16)], kv, mask=bm)
        plsc.store_compressed(bap.at[pl.ds(nb_, 16)], pv, mask=bm)
        return (ns_ + jnp.sum(sm.astype(jnp.int32)),
                nb_ + jnp.sum(bm.astype(jnp.int32)))

    ns, nb1 = lax.fori_loop(0, nvec, cbody, (jnp.int32(0), jnp.int32(0)))

    ns, nb2 = _list_level(3, 1023, bak, bap, nb1, ns, selk, selp,
                          hist, gsum, jnp.int32(K) - ns)
    ns, nb3 = _list_level(0, 7, bak, bap, nb2, ns, selk, selp,
                          hist, gsum, jnp.int32(K) - ns)

    need = jnp.minimum(jnp.int32(K) - ns, nb3)

    def tbody(j, ns_):
        take = (j * 16 + iota) < need
        kv = bak[pl.ds(j * 16, 16)]
        pv = bap[pl.ds(j * 16, 16)]
        plsc.store_compressed(selk.at[pl.ds(ns_, 16)], kv, mask=take)
        plsc.store_compressed(selp.at[pl.ds(ns_, 16)], pv, mask=take)
        return ns_ + jnp.sum(take.astype(jnp.int32))

    ns = lax.fori_loop(0, (need + 15) >> 4, tbody, ns)
    return ns


def _rank_sort(selk, selp, n_sel, skey, spidx):
    iota = _iota()
    zero16 = jnp.zeros((16,), jnp.int32)
    for g in range(13):
        skey[pl.ds(g * 16, 16)] = zero16
        spidx[pl.ds(g * 16, 16)] = zero16
    ng = (n_sel + 15) >> 4
    lane0 = iota == 0

    def jbody(j, carry):
        gj = j >> 4
        lj = j & 15
        kv = selk[pl.ds(gj * 16, 16)]
        pvv = selp[pl.ds(gj * 16, 16)]
        kjv = _splat_lane(kv, lj)
        pjv = _splat_lane(pvv, lj)

        def gbody(g, rankv):
            ks = selk[pl.ds(g * 16, 16)]
            ps = selp[pl.ds(g * 16, 16)]
            validm = (g * 16 + iota) < n_sel
            gt = (ks > kjv) | ((ks == kjv) & (ps < pjv))
            return rankv + (gt & validm).astype(jnp.int32)

        rankv = lax.fori_loop(0, ng, gbody, zero16)
        rsplat = _splat_lane(plsc.cumsum(rankv), jnp.int32(15))
        plsc.store_scatter(skey, [rsplat], kjv, mask=lane0)
        plsc.store_scatter(spidx, [rsplat], pjv, mask=lane0)
        return carry

    lax.fori_loop(0, n_sel, jbody, 0)


def _phase_b_body(keys_hbm, table2, dets_hbm, detb_hbm,
                  sbuf, hist, gsum, selk, selp, bak, bap,
                  skey, spidx, grows, bx1, by1, bx2, by2, barea,
                  supp, detk, dstage, ia, ib, sem):
    iota = _iota()
    zero16 = jnp.zeros((16,), jnp.int32)
    zero16f = jnp.zeros((16,), jnp.float32)

    @pl.loop(0, 1024)
    def _zh(i):
        hist[pl.ds(i * 16, 16)] = zero16

    @pl.loop(0, TPW)
    def _task(t):
        task = _wid() * TPW + t
        b = task // FG
        pltpu.sync_copy(keys_hbm.at[pl.ds(pl.multiple_of(task * P, 16), P)],
                        sbuf)
        n_sel = _select_topk(sbuf, P // 16, hist, gsum, selk, selp,
                             bak, bap)
        _rank_sort(selk, selp, n_sel, skey, spidx)

        base = b * P
        for v in range(8):
            sp = spidx[pl.ds(v * 16, 16)]
            ia[pl.ds(v * 16, 16)] = (base + sp) >> 3
        for v in range(5):
            sp = spidx[pl.ds(128 + v * 16, 16)]
            ib[pl.ds(v * 16, 16)] = (base + sp) >> 3
        d1 = pltpu.async_copy(table2.at[ia], grows.at[pl.ds(0, 128)], sem)
        d2 = pltpu.async_copy(table2.at[ib], grows.at[pl.ds(128, 80)], sem)
        d1.wait()
        d2.wait()

        for g in range(13):
            rg = g * 16 + iota
            sp = spidx[pl.ds(g * 16, 16)]
            colb = (sp & 7) * 16
            comps = [plsc.load_gather(grows, [rg, colb + cc])
                     for cc in range(12)]
            l0, l1, l2, l3, p0, p1, p2, p3, v0, v1, v2, v3 = comps
            cx = p0 + l0 * v0 * p2
            cy = p1 + l1 * v1 * p3
            w = p2 * jnp.exp(l2 * v2)
            h = p3 * jnp.exp(l3 * v3)
            x1 = cx - w / 2.0
            y1 = cy - h / 2.0
            x2 = x1 + w
            y2 = y1 + h
            bx1[pl.ds(g * 16, 16)] = x1
            by1[pl.ds(g * 16, 16)] = y1
            bx2[pl.ds(g * 16, 16)] = x2
            by2[pl.ds(g * 16, 16)] = y2
            barea[pl.ds(g * 16, 16)] = (
                jnp.maximum(x2 - x1, 0.0) * jnp.maximum(y2 - y1, 0.0))
            supp[pl.ds(g * 16, 16)] = zero16f
            detk[pl.ds(g * 16, 16)] = zero16

        ng = (n_sel + 15) >> 4

        def ibody(i, carry):
            gi = i >> 4
            li = i & 15
            si = _extract(supp[pl.ds(gi * 16, 16)], li)
            keep = si == 0.0
            kiv = _splat_lane(skey[pl.ds(gi * 16, 16)], li)
            plsc.store_scatter(detk, [_splat(i)], kiv,
                               mask=(iota == 0) & keep)

            @pl.when(keep)
            def _sweep():
                xx1 = _splat_lane(bx1[pl.ds(gi * 16, 16)], li)
                yy1 = _splat_lane(by1[pl.ds(gi * 16, 16)], li)
                xx2 = _splat_lane(bx2[pl.ds(gi * 16, 16)], li)
                yy2 = _splat_lane(by2[pl.ds(gi * 16, 16)], li)
                aai = _splat_lane(barea[pl.ds(gi * 16, 16)], li)

                def gbody(g, cc):
                    ax1 = bx1[pl.ds(g * 16, 16)]
                    ay1 = by1[pl.ds(g * 16, 16)]
                    ax2 = bx2[pl.ds(g * 16, 16)]
                    ay2 = by2[pl.ds(g * 16, 16)]
                    aar = barea[pl.ds(g * 16, 16)]
                    ix1 = jnp.maximum(ax1, xx1)
                    iy1 = jnp.maximum(ay1, yy1)
                    ix2 = jnp.minimum(ax2, xx2)
                    iy2 = jnp.minimum(ay2, yy2)
                    iw = jnp.maximum(ix2 - ix1, 0.0)
                    ih = jnp.maximum(iy2 - iy1, 0.0)
                    inter = iw * ih
                    union = aar + aai - inter
                    q = inter / jnp.maximum(union, 1e-12)
                    hit = (q > NMS_T).astype(jnp.float32)
                    sg = supp[pl.ds(g * 16, 16)]
                    supp[pl.ds(g * 16, 16)] = jnp.maximum(sg, hit)
                    return cc

                lax.fori_loop(gi, ng, gbody, 0)

            return carry

        lax.fori_loop(0, n_sel, ibody, 0)

        for g in range(13):
            rg = g * 16 + iota
            for comp, ref in ((0, bx1), (1, by1), (2, bx2), (3, by2)):
                plsc.store_scatter(dstage, [rg * 16 + comp],
                                   ref[pl.ds(g * 16, 16)])

        pltpu.sync_copy(detk.at[pl.ds(0, K)],
                        dets_hbm.at[pl.ds(pl.multiple_of(task * K, 8), K)])
        pltpu.sync_copy(
            dstage.at[pl.ds(0, K * 16)],
            detb_hbm.at[pl.ds(pl.multiple_of(task * K * 16, 16), K * 16)])


def _phase_c_body(dets_hbm, detb2, out_hbm,
                  sbuf, hist, gsum, selk, selp, bak, bap,
                  skey, spidx, rowbuf, grows, ia, ib, sem):
    iota = _iota()
    zero16 = jnp.zeros((16,), jnp.int32)
    zero16f = jnp.zeros((16,), jnp.float32)
    wid = _wid()

    @pl.loop(0, 1024)
    def _zh(i):
        hist[pl.ds(i * 16, 16)] = zero16

    @pl.when(wid < B)
    def _img():
        b = wid
        pltpu.sync_copy(
            dets_hbm.at[pl.ds(pl.multiple_of(b * FG * K, 16), FG * K)],
            sbuf)
        n_sel = _select_topk(sbuf, FG * K // 16, hist, gsum, selk, selp,
                             bak, bap)
        _rank_sort(selk, selp, n_sel, skey, spidx)

        base = b * FG * K
        for v in range(8):
            sp = spidx[pl.ds(v * 16, 16)]
            ia[pl.ds(v * 16, 16)] = (base + sp) >> 3
        for v in range(5):
            sp = spidx[pl.ds(128 + v * 16, 16)]
            ib[pl.ds(v * 16, 16)] = (base + sp) >> 3
        d1 = pltpu.async_copy(detb2.at[ia], grows.at[pl.ds(0, 128)], sem)
        d2 = pltpu.async_copy(detb2.at[ib], grows.at[pl.ds(128, 80)], sem)
        d1.wait()
        d2.wait()

        for g in range(88):
            rowbuf[pl.ds(g * 16, 16)] = zero16f

        bf = _splat(jnp.float32(b))
        for g in range(13):
            rg = g * 16 + iota
            valid = rg < n_sel
            kv = skey[pl.ds(g * 16, 16)]
            sp = spidx[pl.ds(g * 16, 16)]
            score = plsc.bitcast(kv, jnp.float32)
            label = (sp // K + 1).astype(jnp.float32)
            r7 = rg * 7
            plsc.store_scatter(rowbuf, [r7], bf, mask=valid)
            plsc.store_scatter(rowbuf, [r7 + 1], label, mask=valid)
            plsc.store_scatter(rowbuf, [r7 + 2], score, mask=valid)
            colb = (sp & 7) * 16
            for comp in range(4):
                bv = plsc.load_gather(grows, [rg, colb + comp])
                plsc.store_scatter(rowbuf, [r7 + 3 + comp], bv, mask=valid)

        pltpu.sync_copy(
            rowbuf.at[pl.ds(0, K * 7)],
            out_hbm.at[pl.ds(pl.multiple_of(b * K * 7, 8), K * 7)])


def kernel(loc_data, conf_data, prior_data):
    conf_flat = conf_data.reshape(B * P * C)
    loc_flat = loc_data.reshape(B * P * 4)
    pv_flat = prior_data.reshape(2 * P * 4)

    phase_a = pl.kernel(
        _phase_a_body,
        out_type=(
            jax.ShapeDtypeStruct((NTASK * P,), jnp.int32),
            jax.ShapeDtypeStruct((B * P * 16,), jnp.float32),
        ),
        mesh=_MESH,
        compiler_params=_CP,
        scratch_types=[
            pltpu.VMEM((400 * C,), jnp.float32),
            pltpu.VMEM((FG * 400,), jnp.int32),
            pltpu.VMEM((752 * 4,), jnp.float32),
            pltpu.VMEM((752 * 4,), jnp.float32),
            pltpu.VMEM((752 * 4,), jnp.float32),
            pltpu.VMEM((752 * 16,), jnp.float32),
            pltpu.SemaphoreType.DMA,
        ],
    )
    keys, table = phase_a(conf_flat, loc_flat, pv_flat)

    phase_b = pl.kernel(
        _phase_b_body,
        out_type=(
            jax.ShapeDtypeStruct((NTASK * K,), jnp.int32),
            jax.ShapeDtypeStruct((NTASK * K * 16,), jnp.float32),
        ),
        mesh=_MESH,
        compiler_params=_CP,
        scratch_types=[
            pltpu.VMEM((P,), jnp.int32),
            pltpu.VMEM((16 * 1024,), jnp.int32),
            pltpu.VMEM((1024,), jnp.int32),
            pltpu.VMEM((256,), jnp.int32),
            pltpu.VMEM((256,), jnp.int32),
            pltpu.VMEM((P,), jnp.int32),
            pltpu.VMEM((P,), jnp.int32),
            pltpu.VMEM((208,), jnp.int32),
            pltpu.VMEM((208,), jnp.int32),
            pltpu.VMEM((208, 128), jnp.float32),
            pltpu.VMEM((208,), jnp.float32),
            pltpu.VMEM((208,), jnp.float32),
            pltpu.VMEM((208,), jnp.float32),
            pltpu.VMEM((208,), jnp.float32),
            pltpu.VMEM((208,), jnp.float32),
            pltpu.VMEM((208,), jnp.float32),
            pltpu.VMEM((208,), jnp.int32),
            pltpu.VMEM((208 * 16,), jnp.float32),
            pltpu.VMEM((128,), jnp.int32),
            pltpu.VMEM((80,), jnp.int32),
            pltpu.SemaphoreType.DMA,
        ],
    )
    dets, detb = phase_b(keys, table.reshape(B * P // 8, 128))

    phase_c = pl.kernel(
        _phase_c_body,
        out_type=jax.ShapeDtypeStruct((B * K * 7,), jnp.float32),
        mesh=_MESH,
        compiler_params=_CP,
        scratch_types=[
            pltpu.VMEM((FG * K,), jnp.int32),
            pltpu.VMEM((16 * 1024,), jnp.int32),
            pltpu.VMEM((1024,), jnp.int32),
            pltpu.VMEM((256,), jnp.int32),
            pltpu.VMEM((256,), jnp.int32),
            pltpu.VMEM((FG * K,), jnp.int32),
            pltpu.VMEM((FG * K,), jnp.int32),
            pltpu.VMEM((208,), jnp.int32),
            pltpu.VMEM((208,), jnp.int32),
            pltpu.VMEM((88 * 16,), jnp.float32),
            pltpu.VMEM((208, 128), jnp.float32),
            pltpu.VMEM((128,), jnp.int32),
            pltpu.VMEM((80,), jnp.int32),
            pltpu.SemaphoreType.DMA,
        ],
    )
    out = phase_c(dets, detb.reshape(NTASK * K // 8, 128))
    return out.reshape(B, 1, K, 7)
